# baseline (speedup 1.0000x reference)
import jax
import jax.numpy as jnp
from jax.experimental import pallas as pl
from jax.experimental.pallas import tpu as pltpu

_GAMMA = 1.0


def _lm_body(y_ref, x_ref, o_ref, m_ref, c_ref, *, bc, ncols, nsteps, nrows):
    c = pl.program_id(0)

    @pl.when(c == 0)
    def _init():
        m_ref[...] = jnp.full((nrows, 1), -jnp.inf, dtype=jnp.float32)
        c_ref[...] = jnp.zeros((nrows, 1), dtype=jnp.float32)

    xb = x_ref[...]
    col_ids = c * bc + jax.lax.broadcasted_iota(jnp.int32, (nrows, bc), 1)
    eq = col_ids == y_ref[...]

    @pl.when(c < nsteps - 1)
    def _main():
        masked = jnp.where(eq, -jnp.inf, xb)
        m_ref[...] = jnp.maximum(
            m_ref[...], jnp.max(masked, axis=1, keepdims=True)
        )
        c_ref[...] = c_ref[...] + jnp.sum(
            jnp.where(eq, xb, 0.0), axis=1, keepdims=True
        )

    @pl.when(c == nsteps - 1)
    def _tail():
        bad = eq | (col_ids >= ncols)
        masked = jnp.where(bad, -jnp.inf, xb)
        m_ref[...] = jnp.maximum(
            m_ref[...], jnp.max(masked, axis=1, keepdims=True)
        )
        c_ref[...] = c_ref[...] + jnp.sum(
            jnp.where(eq, xb, 0.0), axis=1, keepdims=True
        )

    @pl.when(c == nsteps - 1)
    def _fin():
        loss = _GAMMA + m_ref[...] - c_ref[...]
        o_ref[...] = (jnp.sum(loss) * (1.0 / nrows)).reshape(1, 1)


def kernel(x, y):
    nrows, ncols = x.shape
    bc = 4096 if ncols >= 4096 else ncols
    nsteps = pl.cdiv(ncols, bc)
    y2 = y.astype(jnp.int32).reshape(nrows, 1)

    import functools
    body = functools.partial(
        _lm_body, bc=bc, ncols=ncols, nsteps=nsteps, nrows=nrows
    )
    out = pl.pallas_call(
        body,
        grid=(nsteps,),
        in_specs=[
            pl.BlockSpec((nrows, 1), lambda c: (0, 0)),
            pl.BlockSpec((nrows, bc), lambda c: (0, c)),
        ],
        out_specs=pl.BlockSpec((1, 1), lambda c: (0, 0)),
        out_shape=jax.ShapeDtypeStruct((1, 1), jnp.float32),
        scratch_shapes=[
            pltpu.VMEM((nrows, 1), jnp.float32),
            pltpu.VMEM((nrows, 1), jnp.float32),
        ],
        compiler_params=pltpu.CompilerParams(
            dimension_semantics=("arbitrary",),
        ),
    )(y2, x)
    return out[0, 0]


# speedup vs baseline: 1.0088x; 1.0088x over previous
import functools

import jax
import jax.numpy as jnp
from jax.experimental import pallas as pl
from jax.experimental.pallas import tpu as pltpu

_GAMMA = 1.0


def _lm_body(y_ref, x_ref, o_ref, s_ref, *, ncols, nsteps, br, nrows):
    r = pl.program_id(0)

    xb = x_ref[...]
    col_ids = jax.lax.broadcasted_iota(jnp.int32, (br, x_ref.shape[1]), 1)
    eq = col_ids == y_ref[...]
    bad = eq | (col_ids >= ncols)
    masked = jnp.where(bad, -jnp.inf, xb)
    m = jnp.max(masked, axis=1, keepdims=True)
    corr = jnp.sum(jnp.where(eq, xb, 0.0), axis=1, keepdims=True)
    partial = jnp.sum(_GAMMA + m - corr)

    @pl.when(r == 0)
    def _init():
        s_ref[0, 0] = 0.0

    s_ref[0, 0] = s_ref[0, 0] + partial

    @pl.when(r == nsteps - 1)
    def _fin():
        o_ref[0, 0] = s_ref[0, 0] * (1.0 / nrows)


def kernel(x, y):
    nrows, ncols = x.shape
    br = 32 if nrows % 32 == 0 else 8
    nsteps = nrows // br

    body = functools.partial(
        _lm_body, ncols=ncols, nsteps=nsteps, br=br, nrows=nrows
    )
    out = pl.pallas_call(
        body,
        grid=(nsteps,),
        in_specs=[
            pl.BlockSpec((br, 1), lambda r: (r, 0)),
            pl.BlockSpec((br, ncols), lambda r: (r, 0)),
        ],
        out_specs=pl.BlockSpec(memory_space=pltpu.SMEM),
        out_shape=jax.ShapeDtypeStruct((1, 1), jnp.float32),
        scratch_shapes=[
            pltpu.SMEM((1, 1), jnp.float32),
        ],
        compiler_params=pltpu.CompilerParams(
            dimension_semantics=("arbitrary",),
        ),
    )(y.astype(jnp.int32).reshape(nrows, 1), x)
    return out[0, 0]


# speedup vs baseline: 1.0238x; 1.0149x over previous
import functools

import jax
import jax.numpy as jnp
from jax.experimental import pallas as pl
from jax.experimental.pallas import tpu as pltpu

_GAMMA = 1.0
_P = 4


def _lm_body(y_ref, *refs, ncols, nsteps, br, nrows, sw):
    x_refs = refs[:_P]
    o_ref = refs[_P]
    s_ref = refs[_P + 1]
    r = pl.program_id(0)

    yv = y_ref[...]
    m = None
    corr = None
    for p, x_ref in enumerate(x_refs):
        xb = x_ref[...]
        li = jax.lax.broadcasted_iota(jnp.int32, (br, x_ref.shape[1]), 1)
        col_ids = p * sw + li
        eq = col_ids == yv
        wp = min(sw, ncols - p * sw)
        bad = eq | (li >= wp)
        masked = jnp.where(bad, -jnp.inf, xb)
        mp = jnp.max(masked, axis=1, keepdims=True)
        cp = jnp.sum(jnp.where(eq, xb, 0.0), axis=1, keepdims=True)
        m = mp if m is None else jnp.maximum(m, mp)
        corr = cp if corr is None else corr + cp
    partial = jnp.sum(_GAMMA + m - corr)

    @pl.when(r == 0)
    def _init():
        s_ref[0, 0] = 0.0

    s_ref[0, 0] = s_ref[0, 0] + partial

    @pl.when(r == nsteps - 1)
    def _fin():
        o_ref[0, 0] = s_ref[0, 0] * (1.0 / nrows)


def kernel(x, y):
    nrows, ncols = x.shape
    br = 32 if nrows % 32 == 0 else 8
    nsteps = nrows // br
    sw = 128 * pl.cdiv(ncols, _P * 128)

    body = functools.partial(
        _lm_body, ncols=ncols, nsteps=nsteps, br=br, nrows=nrows, sw=sw
    )

    def make_spec(p):
        return pl.BlockSpec((br, sw), lambda r, p=p: (r, p))

    out = pl.pallas_call(
        body,
        grid=(nsteps,),
        in_specs=[pl.BlockSpec((br, 1), lambda r: (r, 0))]
        + [make_spec(p) for p in range(_P)],
        out_specs=pl.BlockSpec(memory_space=pltpu.SMEM),
        out_shape=jax.ShapeDtypeStruct((1, 1), jnp.float32),
        scratch_shapes=[
            pltpu.SMEM((1, 1), jnp.float32),
        ],
        compiler_params=pltpu.CompilerParams(
            dimension_semantics=("arbitrary",),
        ),
    )(y.astype(jnp.int32).reshape(nrows, 1), *([x] * _P))
    return out[0, 0]
